# Initial kernel scaffold; baseline (speedup 1.0000x reference)
import jax
import jax.numpy as jnp
from jax.experimental import pallas as pl


def kernel(x, G_vals, L_vals, F2V_vals, NS, EW, G_cols, L_cols, F2V_cols):
    raise NotImplementedError("write your pallas kernel here")


# speedup vs baseline: 11.0255x; 11.0255x over previous
import functools
import jax
import jax.numpy as jnp
from jax import lax
from jax.experimental import pallas as pl
from jax.experimental.pallas import tpu as pltpu
from jax.experimental.pallas import tpu_sc as plsc

NV = 40962
NF = 81920
B = 2
C = 32
D = B * C

NC = 2
NS = 16
NW = NC * NS

FW = NF // NW
CF = 64
NCHA = FW // CF
RA = CF * 9
UA = 72
NUA = RA // UA

NVP = 43008
VW = NVP // NW
CV = 64
NCHV = VW // CV
RL = CV * 7
UL = 56
NUL = RL // UL
RF = CV * 6
UF = 48
NUF = RF // UF

_mesh = plsc.VectorSubcoreMesh(core_axis_name="c", subcore_axis_name="s")


def _wid():
    return lax.axis_index("s") * NC + lax.axis_index("c")


@functools.partial(
    pl.kernel,
    out_type=[
        jax.ShapeDtypeStruct((NF, D), jnp.float32),
        jax.ShapeDtypeStruct((NF, D), jnp.float32),
    ],
    mesh=_mesh,
    compiler_params=pltpu.CompilerParams(use_tc_tiling_on_sc=False),
    scratch_types=[
        pltpu.VMEM((NUA, UA), jnp.int32),
        pltpu.VMEM((CF, 16), jnp.float32),
        pltpu.VMEM((CF, 16), jnp.float32),
        pltpu.VMEM((CF, 16), jnp.float32),
        pltpu.VMEM((CF, 16), jnp.float32),
        pltpu.VMEM((CF, 16), jnp.float32),
        pltpu.VMEM((RA, D), jnp.float32),
        pltpu.VMEM((CF, D), jnp.float32),
        pltpu.VMEM((CF, D), jnp.float32),
        pltpu.SemaphoreType.DMA,
    ],
)
def _faces_kernel(y_hbm, gc_hbm, gv_hbm, ew_hbm, ns_hbm, ewf_hbm, nsf_hbm,
                  idx_v, gv_v, ew_v, ns_v, we_v, wn_v, rows_v, oew_v, ons_v,
                  sem):
    wid = _wid()

    def chunk(c, carry):
        face0 = pl.multiple_of(wid * FW + c * CF, 8)
        ubase = wid * (FW * 9 // UA) + c * NUA
        pltpu.sync_copy(gc_hbm.at[pl.ds(ubase, NUA)], idx_v)
        pltpu.sync_copy(gv_hbm.at[pl.ds(face0, CF)], gv_v)
        pltpu.sync_copy(ew_hbm.at[pl.ds(face0, CF)], ew_v)
        pltpu.sync_copy(ns_hbm.at[pl.ds(face0, CF)], ns_v)
        cps = [
            pltpu.async_copy(y_hbm.at[idx_v.at[u]],
                             rows_v.at[pl.ds(u * UA, UA)], sem)
            for u in range(NUA)
        ]

        def combine(t, carry2):
            g = gv_v[t]
            we_v[t] = ew_v[t] * g
            wn_v[t] = ns_v[t] * g
            return carry2

        lax.fori_loop(0, CF, combine, 0)
        for cp in cps:
            cp.wait()

        def face(i, carry2):
            we = we_v[i]
            wn = wn_v[i]
            base = i * 9
            acc = [jnp.zeros((16,), jnp.float32) for _ in range(8)]
            for k in range(9):
                r = base + k
                wek = we[k]
                wnk = wn[k]
                for j in range(4):
                    row = rows_v[r, pl.ds(j * 16, 16)]
                    acc[j] = acc[j] + wek * row
                    acc[4 + j] = acc[4 + j] + wnk * row
            for j in range(4):
                oew_v[i, pl.ds(j * 16, 16)] = acc[j]
                ons_v[i, pl.ds(j * 16, 16)] = acc[4 + j]
            return carry2

        lax.fori_loop(0, CF, face, 0)
        pltpu.sync_copy(oew_v, ewf_hbm.at[pl.ds(face0, CF)])
        pltpu.sync_copy(ons_v, nsf_hbm.at[pl.ds(face0, CF)])
        return carry

    lax.fori_loop(0, NCHA, chunk, 0)


@functools.partial(
    pl.kernel,
    out_type=[
        jax.ShapeDtypeStruct((NVP, D), jnp.float32),
        jax.ShapeDtypeStruct((NVP, D), jnp.float32),
        jax.ShapeDtypeStruct((NVP, D), jnp.float32),
    ],
    mesh=_mesh,
    compiler_params=pltpu.CompilerParams(use_tc_tiling_on_sc=False),
    scratch_types=[
        pltpu.VMEM((NUL, UL), jnp.int32),
        pltpu.VMEM((CV, 16), jnp.float32),
        pltpu.VMEM((RL, D), jnp.float32),
        pltpu.VMEM((NUF, UF), jnp.int32),
        pltpu.VMEM((CV, 16), jnp.float32),
        pltpu.VMEM((RF, D), jnp.float32),
        pltpu.VMEM((RF, D), jnp.float32),
        pltpu.VMEM((CV, D), jnp.float32),
        pltpu.VMEM((CV, D), jnp.float32),
        pltpu.VMEM((CV, D), jnp.float32),
        pltpu.SemaphoreType.DMA,
    ],
)
def _verts_kernel(y_hbm, ewf_hbm, nsf_hbm, lc_hbm, lv_hbm, fc_hbm, fv_hbm,
                  lap_hbm, gvew_hbm, gvns_hbm,
                  lidx_v, lval_v, lrows_v, fidx_v, fval_v, erows_v, nrows_v,
                  olap_v, oew_v, ons_v, sem):
    wid = _wid()

    def chunk_l(c, carry):
        row0 = pl.multiple_of(wid * VW + c * CV, 8)
        ubase = wid * (VW * 7 // UL) + c * NUL
        pltpu.sync_copy(lc_hbm.at[pl.ds(ubase, NUL)], lidx_v)
        pltpu.sync_copy(lv_hbm.at[pl.ds(row0, CV)], lval_v)
        cps = [
            pltpu.async_copy(y_hbm.at[lidx_v.at[u]],
                             lrows_v.at[pl.ds(u * UL, UL)], sem)
            for u in range(NUL)
        ]
        for cp in cps:
            cp.wait()

        def vert(i, carry2):
            wl = lval_v[i]
            base = i * 7
            acc = [jnp.zeros((16,), jnp.float32) for _ in range(4)]
            for k in range(7):
                r = base + k
                wk = wl[k]
                for j in range(4):
                    acc[j] = acc[j] + wk * lrows_v[r, pl.ds(j * 16, 16)]
            for j in range(4):
                olap_v[i, pl.ds(j * 16, 16)] = acc[j]
            return carry2

        lax.fori_loop(0, CV, vert, 0)
        pltpu.sync_copy(olap_v, lap_hbm.at[pl.ds(row0, CV)])
        return carry

    def chunk_f(c, carry):
        row0 = pl.multiple_of(wid * VW + c * CV, 8)
        ubase = wid * (VW * 6 // UF) + c * NUF
        pltpu.sync_copy(fc_hbm.at[pl.ds(ubase, NUF)], fidx_v)
        pltpu.sync_copy(fv_hbm.at[pl.ds(row0, CV)], fval_v)
        cps = [
            pltpu.async_copy(ewf_hbm.at[fidx_v.at[u]],
                             erows_v.at[pl.ds(u * UF, UF)], sem)
            for u in range(NUF)
        ] + [
            pltpu.async_copy(nsf_hbm.at[fidx_v.at[u]],
                             nrows_v.at[pl.ds(u * UF, UF)], sem)
            for u in range(NUF)
        ]
        for cp in cps:
            cp.wait()

        def vert(i, carry2):
            wf = fval_v[i]
            base = i * 6
            acc = [jnp.zeros((16,), jnp.float32) for _ in range(8)]
            for k in range(6):
                r = base + k
                wk = wf[k]
                for j in range(4):
                    acc[j] = acc[j] + wk * erows_v[r, pl.ds(j * 16, 16)]
                    acc[4 + j] = acc[4 + j] + wk * nrows_v[r, pl.ds(j * 16, 16)]
            for j in range(4):
                oew_v[i, pl.ds(j * 16, 16)] = acc[j]
                ons_v[i, pl.ds(j * 16, 16)] = acc[4 + j]
            return carry2

        lax.fori_loop(0, CV, vert, 0)
        pltpu.sync_copy(oew_v, gvew_hbm.at[pl.ds(row0, CV)])
        pltpu.sync_copy(ons_v, gvns_hbm.at[pl.ds(row0, CV)])
        return carry

    lax.fori_loop(0, NCHV, chunk_l, 0)
    lax.fori_loop(0, NCHV, chunk_f, 0)


def _pad16(a):
    return jnp.pad(a, ((0, 0), (0, 16 - a.shape[1])))


def kernel(x, G_vals, L_vals, F2V_vals, NS_dir, EW_dir, G_cols, L_cols,
           F2V_cols):
    y = x.reshape(D, NV).T
    gc9 = (G_cols.reshape(3, NF, 3).transpose(1, 0, 2)
           .reshape(NF * 9 // UA, UA))
    gv9 = _pad16(G_vals.reshape(3, NF, 3).transpose(1, 0, 2).reshape(NF, 9))
    ew9 = _pad16(jnp.repeat(EW_dir, 3, axis=1))
    ns9 = _pad16(jnp.repeat(NS_dir, 3, axis=1))

    pad = NVP - NV
    lc = jnp.pad(L_cols, ((0, pad), (0, 0))).reshape(NVP * 7 // UL, UL)
    lv = _pad16(jnp.pad(L_vals, ((0, pad), (0, 0))))
    fc = jnp.pad(F2V_cols, ((0, pad), (0, 0))).reshape(NVP * 6 // UF, UF)
    fv = _pad16(jnp.pad(F2V_vals, ((0, pad), (0, 0))))

    ewf, nsf = _faces_kernel(y, gc9, gv9, ew9, ns9)
    lap, gvew, gvns = _verts_kernel(y, ewf, nsf, lc, lv, fc, fv)

    def back(t):
        return t[:NV].T.reshape(B, C, NV)

    return jnp.stack([x, back(lap), back(gvew), back(gvns)], axis=0)


# speedup vs baseline: 15.0591x; 1.3658x over previous
import functools
import jax
import jax.numpy as jnp
from jax import lax
from jax.experimental import pallas as pl
from jax.experimental.pallas import tpu as pltpu
from jax.experimental.pallas import tpu_sc as plsc

NV = 40962
NF = 81920
B = 2
C = 32
D = B * C

NC = 2
NS = 16
NW = NC * NS

FW = NF // NW
CF = 64
NCHA = FW // CF
RA = CF * 9
UA = 72
NUA = RA // UA

NVP = 43008
VW = NVP // NW
CV = 32
NCHV = VW // CV
RL = CV * 7
UL = 28
NUL = RL // UL
RF = CV * 6
UF = 24
NUF = RF // UF

_mesh = plsc.VectorSubcoreMesh(core_axis_name="c", subcore_axis_name="s")


def _wid():
    return lax.axis_index("s") * NC + lax.axis_index("c")


@functools.partial(
    pl.kernel,
    out_type=[
        jax.ShapeDtypeStruct((NF, D), jnp.float32),
        jax.ShapeDtypeStruct((NF, D), jnp.float32),
    ],
    mesh=_mesh,
    compiler_params=pltpu.CompilerParams(use_tc_tiling_on_sc=False),
    scratch_types=(
        [pltpu.VMEM((NUA, UA), jnp.int32)] * 2 +
        [pltpu.VMEM((CF, 16), jnp.float32)] * 6 +
        [pltpu.VMEM((CF, 16), jnp.float32)] * 2 +
        [pltpu.VMEM((RA, D), jnp.float32)] * 2 +
        [pltpu.VMEM((CF, D), jnp.float32)] * 4 +
        [pltpu.SemaphoreType.DMA] * 6
    ),
)
def _faces_kernel(y_hbm, gc_hbm, gv_hbm, ew_hbm, ns_hbm, ewf_hbm, nsf_hbm,
                  idx0, idx1, gv0, gv1, ew0, ew1, ns0, ns1, we_v, wn_v,
                  rows0, rows1, oew0, oew1, ons0, ons1,
                  s_st0, s_st1, s_rw0, s_rw1, s_out0, s_out1):
    wid = _wid()
    slot = (
        dict(idx=idx0, gv=gv0, ew=ew0, ns=ns0, rows=rows0, oew=oew0,
             ons=ons0, s_st=s_st0, s_rw=s_rw0, s_out=s_out0),
        dict(idx=idx1, gv=gv1, ew=ew1, ns=ns1, rows=rows1, oew=oew1,
             ons=ons1, s_st=s_st1, s_rw=s_rw1, s_out=s_out1),
    )

    def stage(c, s):
        d = slot[s]
        face0 = pl.multiple_of(wid * FW + c * CF, 8)
        ubase = wid * (FW * 9 // UA) + c * NUA
        pltpu.async_copy(gc_hbm.at[pl.ds(ubase, NUA)], d["idx"], d["s_st"])
        pltpu.async_copy(gv_hbm.at[pl.ds(face0, CF)], d["gv"], d["s_st"])
        pltpu.async_copy(ew_hbm.at[pl.ds(face0, CF)], d["ew"], d["s_st"])
        pltpu.async_copy(ns_hbm.at[pl.ds(face0, CF)], d["ns"], d["s_st"])

    def wait_stage(s):
        d = slot[s]
        pltpu.make_async_copy(gc_hbm.at[pl.ds(0, NUA)], d["idx"],
                              d["s_st"]).wait()
        pltpu.make_async_copy(gv_hbm.at[pl.ds(0, CF)], d["gv"],
                              d["s_st"]).wait()
        pltpu.make_async_copy(ew_hbm.at[pl.ds(0, CF)], d["ew"],
                              d["s_st"]).wait()
        pltpu.make_async_copy(ns_hbm.at[pl.ds(0, CF)], d["ns"],
                              d["s_st"]).wait()

    def p1(c, s):
        d = slot[s]
        wait_stage(s)
        for u in range(NUA):
            pltpu.async_copy(y_hbm.at[d["idx"].at[u]],
                             d["rows"].at[pl.ds(u * UA, UA)], d["s_rw"])

    def wait_out(s):
        d = slot[s]
        pltpu.make_async_copy(d["oew"], ewf_hbm.at[pl.ds(0, CF)],
                              d["s_out"]).wait()
        pltpu.make_async_copy(d["ons"], nsf_hbm.at[pl.ds(0, CF)],
                              d["s_out"]).wait()

    def p2(c, s, first=False, last=False):
        d = slot[s]
        for u in range(NUA):
            pltpu.make_async_copy(y_hbm.at[d["idx"].at[u]],
                                  d["rows"].at[pl.ds(u * UA, UA)],
                                  d["s_rw"]).wait()
        if not first:
            wait_out(s)

        def combine(t, carry):
            g = d["gv"][t]
            we_v[t] = d["ew"][t] * g
            wn_v[t] = d["ns"][t] * g
            return carry

        lax.fori_loop(0, CF, combine, 0)
        if not last:
            stage(c + 2, s)

        rows_v, oew_v, ons_v = d["rows"], d["oew"], d["ons"]

        def face(i, carry):
            we = we_v[i]
            wn = wn_v[i]
            base = i * 9
            acc = [jnp.zeros((16,), jnp.float32) for _ in range(8)]
            for k in range(9):
                r = base + k
                wek = we[k]
                wnk = wn[k]
                for j in range(4):
                    row = rows_v[r, pl.ds(j * 16, 16)]
                    acc[j] = acc[j] + wek * row
                    acc[4 + j] = acc[4 + j] + wnk * row
            for j in range(4):
                oew_v[i, pl.ds(j * 16, 16)] = acc[j]
                ons_v[i, pl.ds(j * 16, 16)] = acc[4 + j]
            return carry

        lax.fori_loop(0, CF, face, 0)
        face0 = pl.multiple_of(wid * FW + c * CF, 8)
        pltpu.async_copy(oew_v, ewf_hbm.at[pl.ds(face0, CF)], d["s_out"])
        pltpu.async_copy(ons_v, nsf_hbm.at[pl.ds(face0, CF)], d["s_out"])

    stage(0, 0)
    stage(1, 1)
    p1(0, 0)
    p1(1, 1)
    p2(0, 0, first=True)
    p1(2, 0)
    p2(1, 1, first=True)

    def body(t, carry):
        c = 2 * t
        p1(c + 1, 1)
        p2(c, 0)
        p1(c + 2, 0)
        p2(c + 1, 1)
        return carry

    lax.fori_loop(1, NCHA // 2 - 1, body, 0)
    c = NCHA - 2
    p1(c + 1, 1)
    p2(c, 0, last=True)
    p2(c + 1, 1, last=True)
    wait_out(0)
    wait_out(1)


@functools.partial(
    pl.kernel,
    out_type=[
        jax.ShapeDtypeStruct((NVP, D), jnp.float32),
        jax.ShapeDtypeStruct((NVP, D), jnp.float32),
        jax.ShapeDtypeStruct((NVP, D), jnp.float32),
    ],
    mesh=_mesh,
    compiler_params=pltpu.CompilerParams(use_tc_tiling_on_sc=False),
    scratch_types=(
        [pltpu.VMEM((NUL, UL), jnp.int32)] * 2 +
        [pltpu.VMEM((NUF, UF), jnp.int32)] * 2 +
        [pltpu.VMEM((CV, 16), jnp.float32)] * 4 +
        [pltpu.VMEM((RL, D), jnp.float32)] * 2 +
        [pltpu.VMEM((RF, D), jnp.float32)] * 4 +
        [pltpu.VMEM((CV, D), jnp.float32)] * 6 +
        [pltpu.SemaphoreType.DMA] * 6
    ),
)
def _verts_kernel(y_hbm, ewf_hbm, nsf_hbm, lc_hbm, lv_hbm, fc_hbm, fv_hbm,
                  lap_hbm, gvew_hbm, gvns_hbm,
                  lidx0, lidx1, fidx0, fidx1, lval0, lval1, fval0, fval1,
                  lrows0, lrows1, erows0, erows1, nrows0, nrows1,
                  olap0, olap1, oew0, oew1, ons0, ons1,
                  s_st0, s_st1, s_rw0, s_rw1, s_out0, s_out1):
    wid = _wid()
    slot = (
        dict(lidx=lidx0, fidx=fidx0, lval=lval0, fval=fval0, lrows=lrows0,
             erows=erows0, nrows=nrows0, olap=olap0, oew=oew0, ons=ons0,
             s_st=s_st0, s_rw=s_rw0, s_out=s_out0),
        dict(lidx=lidx1, fidx=fidx1, lval=lval1, fval=fval1, lrows=lrows1,
             erows=erows1, nrows=nrows1, olap=olap1, oew=oew1, ons=ons1,
             s_st=s_st1, s_rw=s_rw1, s_out=s_out1),
    )

    def stage(c, s):
        d = slot[s]
        row0 = pl.multiple_of(wid * VW + c * CV, 8)
        lub = wid * (VW * 7 // UL) + c * NUL
        fub = wid * (VW * 6 // UF) + c * NUF
        pltpu.async_copy(lc_hbm.at[pl.ds(lub, NUL)], d["lidx"], d["s_st"])
        pltpu.async_copy(fc_hbm.at[pl.ds(fub, NUF)], d["fidx"], d["s_st"])
        pltpu.async_copy(lv_hbm.at[pl.ds(row0, CV)], d["lval"], d["s_st"])
        pltpu.async_copy(fv_hbm.at[pl.ds(row0, CV)], d["fval"], d["s_st"])

    def wait_stage(s):
        d = slot[s]
        pltpu.make_async_copy(lc_hbm.at[pl.ds(0, NUL)], d["lidx"],
                              d["s_st"]).wait()
        pltpu.make_async_copy(fc_hbm.at[pl.ds(0, NUF)], d["fidx"],
                              d["s_st"]).wait()
        pltpu.make_async_copy(lv_hbm.at[pl.ds(0, CV)], d["lval"],
                              d["s_st"]).wait()
        pltpu.make_async_copy(fv_hbm.at[pl.ds(0, CV)], d["fval"],
                              d["s_st"]).wait()

    def p1(c, s):
        d = slot[s]
        wait_stage(s)
        for u in range(NUL):
            pltpu.async_copy(y_hbm.at[d["lidx"].at[u]],
                             d["lrows"].at[pl.ds(u * UL, UL)], d["s_rw"])
        for u in range(NUF):
            pltpu.async_copy(ewf_hbm.at[d["fidx"].at[u]],
                             d["erows"].at[pl.ds(u * UF, UF)], d["s_rw"])
            pltpu.async_copy(nsf_hbm.at[d["fidx"].at[u]],
                             d["nrows"].at[pl.ds(u * UF, UF)], d["s_rw"])

    def wait_out(s):
        d = slot[s]
        pltpu.make_async_copy(d["olap"], lap_hbm.at[pl.ds(0, CV)],
                              d["s_out"]).wait()
        pltpu.make_async_copy(d["oew"], gvew_hbm.at[pl.ds(0, CV)],
                              d["s_out"]).wait()
        pltpu.make_async_copy(d["ons"], gvns_hbm.at[pl.ds(0, CV)],
                              d["s_out"]).wait()

    def p2(c, s, first=False, last=False):
        d = slot[s]
        for u in range(NUL):
            pltpu.make_async_copy(y_hbm.at[d["lidx"].at[u]],
                                  d["lrows"].at[pl.ds(u * UL, UL)],
                                  d["s_rw"]).wait()
        for u in range(NUF):
            pltpu.make_async_copy(ewf_hbm.at[d["fidx"].at[u]],
                                  d["erows"].at[pl.ds(u * UF, UF)],
                                  d["s_rw"]).wait()
            pltpu.make_async_copy(nsf_hbm.at[d["fidx"].at[u]],
                                  d["nrows"].at[pl.ds(u * UF, UF)],
                                  d["s_rw"]).wait()
        if not first:
            wait_out(s)
        lval_v, fval_v = d["lval"], d["fval"]
        lrows_v, erows_v, nrows_v = d["lrows"], d["erows"], d["nrows"]
        olap_v, oew_v, ons_v = d["olap"], d["oew"], d["ons"]

        def vert(i, carry):
            wl = lval_v[i]
            base = i * 7
            acc = [jnp.zeros((16,), jnp.float32) for _ in range(4)]
            for k in range(7):
                r = base + k
                wk = wl[k]
                for j in range(4):
                    acc[j] = acc[j] + wk * lrows_v[r, pl.ds(j * 16, 16)]
            for j in range(4):
                olap_v[i, pl.ds(j * 16, 16)] = acc[j]
            return carry

        lax.fori_loop(0, CV, vert, 0)

        def vert2(i, carry):
            wf = fval_v[i]
            base = i * 6
            acc = [jnp.zeros((16,), jnp.float32) for _ in range(8)]
            for k in range(6):
                r = base + k
                wk = wf[k]
                for j in range(4):
                    acc[j] = acc[j] + wk * erows_v[r, pl.ds(j * 16, 16)]
                    acc[4 + j] = acc[4 + j] + wk * nrows_v[r, pl.ds(j * 16, 16)]
            for j in range(4):
                oew_v[i, pl.ds(j * 16, 16)] = acc[j]
                ons_v[i, pl.ds(j * 16, 16)] = acc[4 + j]
            return carry

        lax.fori_loop(0, CV, vert2, 0)
        row0 = pl.multiple_of(wid * VW + c * CV, 8)
        pltpu.async_copy(olap_v, lap_hbm.at[pl.ds(row0, CV)], d["s_out"])
        pltpu.async_copy(oew_v, gvew_hbm.at[pl.ds(row0, CV)], d["s_out"])
        pltpu.async_copy(ons_v, gvns_hbm.at[pl.ds(row0, CV)], d["s_out"])
        if not last:
            stage(c + 2, s)

    stage(0, 0)
    stage(1, 1)
    p1(0, 0)
    p1(1, 1)
    p2(0, 0, first=True)
    p1(2, 0)
    p2(1, 1, first=True)

    def body(t, carry):
        c = 2 * t
        p1(c + 1, 1)
        p2(c, 0)
        p1(c + 2, 0)
        p2(c + 1, 1)
        return carry

    lax.fori_loop(1, NCHV // 2 - 1, body, 0)
    c = NCHV - 2
    p1(c + 1, 1)
    p2(c, 0, last=True)
    p2(c + 1, 1, last=True)
    wait_out(0)
    wait_out(1)


def _pad16(a):
    return jnp.pad(a, ((0, 0), (0, 16 - a.shape[1])))


def kernel(x, G_vals, L_vals, F2V_vals, NS_dir, EW_dir, G_cols, L_cols,
           F2V_cols):
    y = x.reshape(D, NV).T
    gc9 = (G_cols.reshape(3, NF, 3).transpose(1, 0, 2)
           .reshape(NF * 9 // UA, UA))
    gv9 = _pad16(G_vals.reshape(3, NF, 3).transpose(1, 0, 2).reshape(NF, 9))
    ew9 = _pad16(jnp.repeat(EW_dir, 3, axis=1))
    ns9 = _pad16(jnp.repeat(NS_dir, 3, axis=1))

    pad = NVP - NV
    lc = jnp.pad(L_cols, ((0, pad), (0, 0))).reshape(NVP * 7 // UL, UL)
    lv = _pad16(jnp.pad(L_vals, ((0, pad), (0, 0))))
    fc = jnp.pad(F2V_cols, ((0, pad), (0, 0))).reshape(NVP * 6 // UF, UF)
    fv = _pad16(jnp.pad(F2V_vals, ((0, pad), (0, 0))))

    ewf, nsf = _faces_kernel(y, gc9, gv9, ew9, ns9)
    lap, gvew, gvns = _verts_kernel(y, ewf, nsf, lc, lv, fc, fv)

    def back(t):
        return t[:NV].T.reshape(B, C, NV)

    return jnp.stack([x, back(lap), back(gvew), back(gvns)], axis=0)
